# Initial kernel scaffold; baseline (speedup 1.0000x reference)
import jax
import jax.numpy as jnp
from jax.experimental import pallas as pl


def kernel(x, W):
    raise NotImplementedError("write your pallas kernel here")


# speedup vs baseline: 3.9400x; 3.9400x over previous
import functools

import jax
import jax.numpy as jnp
from jax import lax
from jax.experimental import pallas as pl
from jax.experimental.pallas import tpu as pltpu
from jax.experimental.pallas import tpu_sc as plsc

_C_IN = 1000
_D = 64
_BATCH = 16384
_HIST = 200

_NC = 2
_NS = 16
_NW = _NC * _NS

_B = _BATCH * _HIST
_ROWS_PER_W = _B // _NW
_GATHER = 128
_KPG = 4
_GROUP = _GATHER * _KPG
_NGROUPS = _ROWS_PER_W // _GROUP


def _sc_body(x_hbm, w_hbm, out_hbm, idx_v, rows_v, sem):
    wid = lax.axis_index("s") * _NC + lax.axis_index("c")

    def group(g, carry):
        pltpu.sync_copy(x_hbm.at[wid, g], idx_v)
        for k in range(_KPG):
            pltpu.async_copy(
                w_hbm.at[idx_v.at[k]],
                rows_v.at[pl.ds(k * _GATHER, _GATHER)],
                sem,
            ).wait()
        pltpu.sync_copy(rows_v, out_hbm.at[wid, g])
        return carry

    lax.fori_loop(0, _NGROUPS, group, 0, unroll=False)


@jax.jit
def _lookup(x32, w):
    mesh = plsc.VectorSubcoreMesh(
        core_axis_name="c", subcore_axis_name="s",
        num_cores=_NC, num_subcores=_NS,
    )
    run = pl.kernel(
        _sc_body,
        out_type=jax.ShapeDtypeStruct((_NW, _NGROUPS, _GROUP, _D), jnp.float32),
        mesh=mesh,
        scratch_types=[
            pltpu.VMEM((_KPG, _GATHER), jnp.int32),
            pltpu.VMEM((_GROUP, _D), jnp.float32),
            pltpu.SemaphoreType.DMA,
        ],
        compiler_params=pltpu.CompilerParams(use_tc_tiling_on_sc=False),
    )
    return run(x32, w)


def kernel(x, W):
    x32 = x.reshape(-1).astype(jnp.int32).reshape(_NW, _NGROUPS, _KPG, _GATHER)
    out = _lookup(x32, W)
    return lax.stop_gradient(out.reshape(_BATCH, _HIST, _D))


# speedup vs baseline: 4.1542x; 1.0544x over previous
import functools

import jax
import jax.numpy as jnp
from jax import lax
from jax.experimental import pallas as pl
from jax.experimental.pallas import tpu as pltpu
from jax.experimental.pallas import tpu_sc as plsc

_C_IN = 1000
_D = 64
_BATCH = 16384
_HIST = 200

_NC = 2
_NS = 16
_NW = _NC * _NS

_B = _BATCH * _HIST
_ROWS_PER_W = _B // _NW
_GATHER = 128
_KPG = 4
_GROUP = _GATHER * _KPG
_NGROUPS = _ROWS_PER_W // _GROUP


def _sc_body(x_hbm, w_hbm, out_hbm, idx_v, rows_v,
             sem_i0, sem_i1, sem_g0, sem_g1, sem_o0, sem_o1):
    wid = lax.axis_index("s") * _NC + lax.axis_index("c")
    sem_i = (sem_i0, sem_i1)
    sem_g = (sem_g0, sem_g1)
    sem_o = (sem_o0, sem_o1)

    def start_idx(buf, g):
        pltpu.make_async_copy(x_hbm.at[wid, g], idx_v.at[buf], sem_i[buf]).start()

    def wait_idx(buf):
        pltpu.make_async_copy(x_hbm.at[wid, 0], idx_v.at[buf], sem_i[buf]).wait()

    def start_gathers(buf):
        for k in range(_KPG):
            pltpu.make_async_copy(
                w_hbm.at[idx_v.at[buf, k]],
                rows_v.at[buf, pl.ds(k * _GATHER, _GATHER)],
                sem_g[buf],
            ).start()

    def wait_gathers(buf):
        for k in range(_KPG):
            pltpu.make_async_copy(
                w_hbm.at[idx_v.at[buf, k]],
                rows_v.at[buf, pl.ds(k * _GATHER, _GATHER)],
                sem_g[buf],
            ).wait()

    def start_out(buf, g):
        pltpu.make_async_copy(rows_v.at[buf], out_hbm.at[wid, g], sem_o[buf]).start()

    def wait_out(buf):
        pltpu.make_async_copy(rows_v.at[buf], out_hbm.at[wid, 0], sem_o[buf]).wait()

    start_idx(0, 0)
    start_idx(1, 1)
    for buf in (0, 1):
        wait_idx(buf)
        start_gathers(buf)
        wait_gathers(buf)
        start_idx(buf, buf + 2)
        start_out(buf, buf)

    def pair(p, carry):
        g0 = 2 * p
        for buf in (0, 1):
            g = g0 + buf
            wait_out(buf)
            wait_idx(buf)
            start_gathers(buf)
            wait_gathers(buf)
            start_idx(buf, lax.rem(g + 2, _NGROUPS))
            start_out(buf, g)
        return carry

    lax.fori_loop(1, _NGROUPS // 2, pair, 0, unroll=False)

    for buf in (0, 1):
        wait_out(buf)
        wait_idx(buf)


@jax.jit
def _lookup(x32, w):
    mesh = plsc.VectorSubcoreMesh(
        core_axis_name="c", subcore_axis_name="s",
        num_cores=_NC, num_subcores=_NS,
    )
    run = pl.kernel(
        _sc_body,
        out_type=jax.ShapeDtypeStruct((_NW, _NGROUPS, _GROUP, _D), jnp.float32),
        mesh=mesh,
        scratch_types=[
            pltpu.VMEM((2, _KPG, _GATHER), jnp.int32),
            pltpu.VMEM((2, _GROUP, _D), jnp.float32),
            pltpu.SemaphoreType.DMA,
            pltpu.SemaphoreType.DMA,
            pltpu.SemaphoreType.DMA,
            pltpu.SemaphoreType.DMA,
            pltpu.SemaphoreType.DMA,
            pltpu.SemaphoreType.DMA,
        ],
        compiler_params=pltpu.CompilerParams(use_tc_tiling_on_sc=False),
    )
    return run(x32, w)


def kernel(x, W):
    x32 = x.reshape(-1).astype(jnp.int32).reshape(_NW, _NGROUPS, _KPG, _GATHER)
    out = _lookup(x32, W)
    return lax.stop_gradient(out.reshape(_BATCH, _HIST, _D))
